# baseline (speedup 1.0000x reference)
import functools

import jax
import jax.numpy as jnp
from jax.experimental import pallas as pl
from jax.experimental.pallas import tpu as pltpu
from jax.experimental.pallas import tpu_sc as plsc

EMBED_DIM = 1024
NUM_EXPERT = 16
TOP_K = 2
EXPERT_DIM = 256
HEAD_DIM = 64
NUM_HEADS = EXPERT_DIM // HEAD_DIM
MAX_POS = 64
SCALING = HEAD_DIM ** (-0.25)


_SC_TILES = 32
_SC_LANES = 16


def _gate_logits_kernel(x_ref, wg_ref, lg_ref, *, tok_per_tile):
    logits = x_ref[...] @ wg_ref[...]
    lt = logits.T
    for t in range(lt.shape[1] // tok_per_tile):
        lg_ref[t] = lt[:, t * tok_per_tile:(t + 1) * tok_per_tile]


def _sc_gate_kernel(lg_hbm, i1_hbm, i2_hbm, g1_hbm, g2_hbm,
                    lt_v, i1_v, i2_v, g1_v, g2_v, *, tok_per_tile):
    w = jax.lax.axis_index("s") * 2 + jax.lax.axis_index("c")
    pltpu.sync_copy(lg_hbm.at[w], lt_v)
    for g in range(tok_per_tile // _SC_LANES):
        sl = pl.ds(g * _SC_LANES, _SC_LANES)
        le = [lt_v[e, sl] for e in range(NUM_EXPERT)]
        m1 = le[0]
        for e in range(1, NUM_EXPERT):
            m1 = jnp.maximum(m1, le[e])
        i1 = jnp.full((_SC_LANES,), NUM_EXPERT, jnp.int32)
        for e in range(NUM_EXPERT):
            i1 = jnp.minimum(i1, jnp.where(le[e] == m1, jnp.int32(e),
                                           jnp.int32(NUM_EXPERT)))
        lm = [jnp.where(i1 == e, -jnp.inf, le[e]) for e in range(NUM_EXPERT)]
        m2 = lm[0]
        for e in range(1, NUM_EXPERT):
            m2 = jnp.maximum(m2, lm[e])
        i2 = jnp.full((_SC_LANES,), NUM_EXPERT, jnp.int32)
        for e in range(NUM_EXPERT):
            i2 = jnp.minimum(i2, jnp.where(lm[e] == m2, jnp.int32(e),
                                           jnp.int32(NUM_EXPERT)))
        g1 = 1.0 / (1.0 + jnp.exp(m2 - m1))
        i1_v[sl] = i1
        i2_v[sl] = i2
        g1_v[sl] = g1
        g2_v[sl] = 1.0 - g1
    dst = pl.ds(w * tok_per_tile, tok_per_tile)
    pltpu.sync_copy(i1_v, i1_hbm.at[dst])
    pltpu.sync_copy(i2_v, i2_hbm.at[dst])
    pltpu.sync_copy(g1_v, g1_hbm.at[dst])
    pltpu.sync_copy(g2_v, g2_hbm.at[dst])


def _qproj_kernel(x_ref, wq_ref, idx_ref, q_ref):
    e = pl.program_id(1)

    @pl.when(e == 0)
    def _():
        q_ref[...] = jnp.zeros_like(q_ref)

    p = ((x_ref[...] @ wq_ref[0]) * SCALING).astype(jnp.bfloat16)
    idx = idx_ref[...]
    for k in range(TOP_K):
        sel = idx[:, k:k + 1] == e
        q_ref[:, k * EXPERT_DIM:(k + 1) * EXPERT_DIM] += jnp.where(
            sel, p, jnp.bfloat16(0))


def _kv_kernel(xk_ref, xv_ref, wk_ref, wv_ref, k_ref, v_ref):
    k_ref[...] = ((xk_ref[...] @ wk_ref[...]) * SCALING).astype(jnp.bfloat16)
    v_ref[...] = (xv_ref[...] @ wv_ref[...]).astype(jnp.bfloat16)


_BC = 512
_BW = 1024


def _attn_kernel(q_ref, k_ref, v_ref, e_ref, y_ref, p_scr, d_scr, *, block_t):
    S = k_ref.shape[0]
    t = pl.program_id(1)
    i0 = t * block_t
    jw = pl.multiple_of(jnp.clip((i0 - 128) // _BC * _BC, 0, S - _BW), _BC)
    rr = jax.lax.broadcasted_iota(jnp.int32, (block_t, _BW), 0)
    cc = jax.lax.broadcasted_iota(jnp.int32, (block_t, _BW), 1)
    d_scr[...] = jnp.clip((jw + cc) - (i0 + rr),
                          1 - MAX_POS, MAX_POS - 1) + MAX_POS
    kk = k_ref[...]
    vv = v_ref[...]
    f32 = jnp.float32
    for h in range(NUM_HEADS):
        qh = q_ref[:, h * HEAD_DIM:(h + 1) * HEAD_DIM]
        kh = kk[:, h * HEAD_DIM:(h + 1) * HEAD_DIM]
        vh = vv[:, h * HEAD_DIM:(h + 1) * HEAD_DIM]
        rlog = jax.lax.dot_general(
            qh, e_ref[h], (((1,), (0,)), ((), ())),
            preferred_element_type=f32)
        rl1 = rlog[:, 1:2]
        rl127 = rlog[:, 127:128]
        rmax = jnp.max(rlog[:, 1:], axis=1, keepdims=True)
        qh32 = qh.astype(f32)
        qn2 = jnp.sum(qh32 * qh32, axis=1, keepdims=True)
        kh32 = kh.astype(f32)
        kn2 = jnp.sum(kh32 * kh32, axis=1, keepdims=True)
        m2 = jnp.sqrt(qn2 * jnp.max(kn2)) + rmax - rl1
        l = None
        for c in range(S // _BC):
            ks = kh[c * _BC:(c + 1) * _BC, :]
            s = jax.lax.dot_general(
                qh, ks, (((1,), (1,)), ((), ())),
                preferred_element_type=f32)
            base = jnp.where(c * _BC >= jw + _BW, rl127 - rl1, 0.0)
            p = jnp.exp(s + (base - m2))
            lc = jnp.sum(p, axis=1, keepdims=True)
            l = lc if l is None else l + lc
            p_scr[:, c * _BC:(c + 1) * _BC] = p.astype(jnp.bfloat16)
        delta = jnp.take_along_axis(rlog, d_scr[...], axis=1) - rl1
        pold = p_scr[:, pl.ds(jw, _BW)].astype(f32)
        pnew = pold * jnp.exp(delta)
        l = l + jnp.sum(pnew - pold, axis=1, keepdims=True)
        p_scr[:, pl.ds(jw, _BW)] = pnew.astype(jnp.bfloat16)
        pv = jax.lax.dot_general(
            p_scr[...], vh, (((1,), (0,)), ((), ())),
            preferred_element_type=f32)
        y_ref[:, h * HEAD_DIM:(h + 1) * HEAD_DIM] = pv / l


def _oproj_kernel(y_ref, idx_ref, gate_ref, wo_ref, o_ref):
    e = pl.program_id(1)

    @pl.when(e == 0)
    def _():
        o_ref[...] = jnp.zeros_like(o_ref)

    idx = idx_ref[...]
    g = gate_ref[...]
    z = None
    for k in range(TOP_K):
        w = jnp.where(idx[:, k:k + 1] == e, g[:, k:k + 1], 0.0)
        zk = y_ref[:, k * EXPERT_DIM:(k + 1) * EXPERT_DIM] * w
        z = zk if z is None else z + zk
    o_ref[...] += z @ wo_ref[0]


def kernel(query, key, value, Wg, Wq, Wk, Wv, Wo, rel_pos_emb):
    T, B, D = query.shape
    S = key.shape[0]
    n = T * B
    x = query.reshape(n, D)
    xk = key.reshape(S * B, D)
    xv = value.reshape(S * B, D)
    f32 = jnp.float32
    bf16 = jnp.bfloat16

    bT = 512
    tok_per_tile = n // _SC_TILES
    tiles_per_blk = bT // tok_per_tile
    lg = pl.pallas_call(
        functools.partial(_gate_logits_kernel, tok_per_tile=tok_per_tile),
        grid=(n // bT,),
        in_specs=[
            pl.BlockSpec((bT, D), lambda i: (i, 0)),
            pl.BlockSpec((D, NUM_EXPERT), lambda i: (0, 0)),
        ],
        out_specs=pl.BlockSpec((tiles_per_blk, NUM_EXPERT, tok_per_tile),
                               lambda i: (i, 0, 0)),
        out_shape=jax.ShapeDtypeStruct((_SC_TILES, NUM_EXPERT, tok_per_tile),
                                       f32),
    )(x, Wg)

    i1, i2, g1, g2 = pl.kernel(
        functools.partial(_sc_gate_kernel, tok_per_tile=tok_per_tile),
        out_type=[
            jax.ShapeDtypeStruct((n,), jnp.int32),
            jax.ShapeDtypeStruct((n,), jnp.int32),
            jax.ShapeDtypeStruct((n,), f32),
            jax.ShapeDtypeStruct((n,), f32),
        ],
        mesh=plsc.VectorSubcoreMesh(core_axis_name="c", subcore_axis_name="s"),
        scratch_types=[
            pltpu.VMEM((NUM_EXPERT, tok_per_tile), f32),
            pltpu.VMEM((tok_per_tile,), jnp.int32),
            pltpu.VMEM((tok_per_tile,), jnp.int32),
            pltpu.VMEM((tok_per_tile,), f32),
            pltpu.VMEM((tok_per_tile,), f32),
        ],
    )(lg)
    idx = jnp.stack([i1, i2], axis=1)
    gates = jnp.stack([g1, g2], axis=1)

    q = pl.pallas_call(
        _qproj_kernel,
        grid=(n // bT, NUM_EXPERT),
        in_specs=[
            pl.BlockSpec((bT, D), lambda i, e: (i, 0)),
            pl.BlockSpec((1, D, EXPERT_DIM), lambda i, e: (e, 0, 0)),
            pl.BlockSpec((bT, TOP_K), lambda i, e: (i, 0)),
        ],
        out_specs=pl.BlockSpec((bT, TOP_K * EXPERT_DIM), lambda i, e: (i, 0)),
        out_shape=jax.ShapeDtypeStruct((n, TOP_K * EXPERT_DIM), bf16),
    )(x, Wq, idx)

    kp, vp = pl.pallas_call(
        _kv_kernel,
        grid=(S * B // bT,),
        in_specs=[
            pl.BlockSpec((bT, D), lambda i: (i, 0)),
            pl.BlockSpec((bT, D), lambda i: (i, 0)),
            pl.BlockSpec((D, EXPERT_DIM), lambda i: (0, 0)),
            pl.BlockSpec((D, EXPERT_DIM), lambda i: (0, 0)),
        ],
        out_specs=[
            pl.BlockSpec((bT, EXPERT_DIM), lambda i: (i, 0)),
            pl.BlockSpec((bT, EXPERT_DIM), lambda i: (i, 0)),
        ],
        out_shape=[
            jax.ShapeDtypeStruct((S * B, EXPERT_DIM), bf16),
            jax.ShapeDtypeStruct((S * B, EXPERT_DIM), bf16),
        ],
    )(xk, xv, Wk, Wv)

    rpe = rel_pos_emb[:, :, :2 * MAX_POS].astype(bf16)

    bA = 256
    y = pl.pallas_call(
        functools.partial(_attn_kernel, block_t=bA),
        grid=(TOP_K, T // bA),
        in_specs=[
            pl.BlockSpec((bA, EXPERT_DIM), lambda k, t: (t, k)),
            pl.BlockSpec((S, EXPERT_DIM), lambda k, t: (0, 0)),
            pl.BlockSpec((S, EXPERT_DIM), lambda k, t: (0, 0)),
            pl.BlockSpec((NUM_HEADS, HEAD_DIM, 2 * MAX_POS),
                         lambda k, t: (0, 0, 0)),
        ],
        out_specs=pl.BlockSpec((bA, EXPERT_DIM), lambda k, t: (t, k)),
        out_shape=jax.ShapeDtypeStruct((n, TOP_K * EXPERT_DIM), f32),
        scratch_shapes=[
            pltpu.VMEM((bA, S), jnp.bfloat16),
            pltpu.VMEM((bA, _BW), jnp.int32),
        ],
    )(q, kp, vp, rpe)

    out = pl.pallas_call(
        _oproj_kernel,
        grid=(n // bT, NUM_EXPERT),
        in_specs=[
            pl.BlockSpec((bT, TOP_K * EXPERT_DIM), lambda i, e: (i, 0)),
            pl.BlockSpec((bT, TOP_K), lambda i, e: (i, 0)),
            pl.BlockSpec((bT, TOP_K), lambda i, e: (i, 0)),
            pl.BlockSpec((1, EXPERT_DIM, D), lambda i, e: (e, 0, 0)),
        ],
        out_specs=pl.BlockSpec((bT, D), lambda i, e: (i, 0)),
        out_shape=jax.ShapeDtypeStruct((n, D), f32),
    )(y, idx, gates, Wo)

    return out.reshape(T, B, D)
